# baseline (speedup 1.0000x reference)
import functools

import jax
import jax.numpy as jnp
from jax import lax
from jax.experimental import pallas as pl
from jax.experimental.pallas import tpu as pltpu
from jax.experimental.pallas import tpu_sc as plsc

N_ROWS = 100000
D = 64
B = 16384 * 200
NW = 32
BPW = B // NW
C = 512
NCHUNK = BPW // C

_mesh = plsc.VectorSubcoreMesh(core_axis_name="c", subcore_axis_name="s")


@functools.partial(
    pl.kernel,
    out_type=jax.ShapeDtypeStruct((B, D), jnp.float32),
    mesh=_mesh,
    scratch_types=[
        pltpu.VMEM((C,), jnp.int32),
        pltpu.VMEM((C, D), jnp.float32),
        pltpu.SemaphoreType.DMA,
    ],
    compiler_params=pltpu.CompilerParams(use_tc_tiling_on_sc=False),
)
def _gather_kernel(idx_hbm, table_hbm, out_hbm, idx_v, rows_v, sem):
    wid = lax.axis_index("s") * 2 + lax.axis_index("c")
    base = wid * BPW

    def body(g, carry):
        off = base + g * C
        pltpu.sync_copy(idx_hbm.at[pl.ds(off, C)], idx_v)
        pltpu.async_copy(table_hbm.at[idx_v], rows_v, sem).wait()
        pltpu.sync_copy(rows_v, out_hbm.at[pl.ds(off, C)])
        return carry

    lax.fori_loop(0, NCHUNK, body, 0)


def kernel(action_idx, embedding_weight):
    idx_flat = action_idx.reshape(-1).astype(jnp.int32)
    out = _gather_kernel(idx_flat, embedding_weight)
    return out.reshape(action_idx.shape + (D,))


# speedup vs baseline: 1.4493x; 1.4493x over previous
import functools

import jax
import jax.numpy as jnp
from jax import lax
from jax.experimental import pallas as pl
from jax.experimental.pallas import tpu as pltpu
from jax.experimental.pallas import tpu_sc as plsc

N_ROWS = 100000
D = 64
DP = 128
B = 16384 * 200
NW = 32
BPW = B // NW
C = 400
NCHUNK = BPW // C
KMAX = NCHUNK // 2

_mesh = plsc.VectorSubcoreMesh(core_axis_name="c", subcore_axis_name="s")


@functools.partial(
    pl.kernel,
    out_type=jax.ShapeDtypeStruct((B, DP), jnp.float32),
    mesh=_mesh,
    scratch_types=[
        pltpu.VMEM((C,), jnp.int32),
        pltpu.VMEM((C,), jnp.int32),
        pltpu.VMEM((C, DP), jnp.float32),
        pltpu.VMEM((C, DP), jnp.float32),
        pltpu.SemaphoreType.DMA,
        pltpu.SemaphoreType.DMA,
        pltpu.SemaphoreType.DMA,
        pltpu.SemaphoreType.DMA,
    ],
)
def _gather_kernel(idx_hbm, table_hbm, out_hbm, idx_a, idx_b,
                   rows_a, rows_b, gsem_a, gsem_b, osem_a, osem_b):
    wid = lax.axis_index("s") * 2 + lax.axis_index("c")
    base = wid * BPW
    idx_c = (idx_a, idx_b)
    rows = (rows_a, rows_b)
    gsem = (gsem_a, gsem_b)
    osem = (osem_a, osem_b)

    def start_gather(i, b):
        pltpu.sync_copy(idx_hbm.at[pl.ds(base + i * C, C)], idx_c[b])
        pltpu.async_copy(table_hbm.at[idx_c[b]], rows[b], gsem[b])

    def wait_gather(b):
        pltpu.make_async_copy(table_hbm.at[idx_c[b]], rows[b],
                              gsem[b]).wait()

    def start_out(i, b):
        pltpu.async_copy(rows[b], out_hbm.at[pl.ds(base + i * C, C)],
                         osem[b])

    def wait_out(i, b):
        pltpu.make_async_copy(rows[b], out_hbm.at[pl.ds(base + i * C, C)],
                              osem[b]).wait()

    start_gather(0, 0)
    start_gather(1, 1)

    def body(k, carry):
        i0 = 2 * k
        wait_gather(0)
        start_out(i0, 0)
        wait_gather(1)
        start_out(i0 + 1, 1)
        wait_out(i0, 0)
        start_gather(i0 + 2, 0)
        wait_out(i0 + 1, 1)
        start_gather(i0 + 3, 1)
        return carry

    lax.fori_loop(0, KMAX - 1, body, 0)
    wait_gather(0)
    start_out(NCHUNK - 2, 0)
    wait_gather(1)
    start_out(NCHUNK - 1, 1)
    wait_out(NCHUNK - 2, 0)
    wait_out(NCHUNK - 1, 1)


def kernel(action_idx, embedding_weight):
    idx_flat = action_idx.reshape(-1).astype(jnp.int32)
    table_pad = jnp.pad(embedding_weight, ((0, 0), (0, DP - D)))
    out = _gather_kernel(idx_flat, table_pad)
    return out[:, :D].reshape(action_idx.shape + (D,))
